# baseline (speedup 1.0000x reference)
import jax
import jax.numpy as jnp
from jax.experimental import pallas as pl

B, N, H = 20, 100, 64
P2 = N // 2
L = 2 * H
NUM_LAYERS = 3
EPS = 1e-5


def _mm(a3, w):
    r, c, k = a3.shape
    out = jax.lax.dot_general(a3.reshape(r * c, k), w,
                              (((1,), (0,)), ((), ())),
                              preferred_element_type=jnp.float32)
    return out.reshape(r, c, out.shape[-1])


def _mm2(a2, w):
    return jax.lax.dot_general(a2, w, (((1,), (0,)), ((), ())),
                               preferred_element_type=jnp.float32)


def _fold(v):
    return v[..., :H] + v[..., H:]


def _tile2(v):
    return jnp.concatenate([v, v], axis=-1)


def _layer_start(e_pk, x_cur, x_pk, eU_w2, eU_b2, eV_wd, eV_w2, eV_b2,
                 nU_w, nU_b, nV_w2, nV_b2):
    Vx_row = _mm2(x_cur, eV_wd) + eV_b2
    Vx_col = _mm2(x_pk, eV_w2) + eV_b2
    Ue = _mm(e_pk, eU_w2) + eU_b2[None]
    e_tmp = Ue + Vx_row[:, None, :] + Vx_col[None, :, :]
    gate = jax.nn.sigmoid(e_tmp)
    Vx2 = _mm2(x_pk, nV_w2) + nV_b2
    Ux = _mm2(x_cur, nU_w) + nU_b
    num = _fold(jnp.sum(gate * Vx2[None, :, :], axis=1))
    den = _fold(jnp.sum(gate, axis=1))
    x_tmp = Ux + num / (1e-20 + den)
    esum = _fold(jnp.sum(e_tmp, axis=(0, 1)))[None, :]
    esq = _fold(jnp.sum(e_tmp * e_tmp, axis=(0, 1)))[None, :]
    xsum = jnp.sum(x_tmp, axis=0)[None, :]
    xsq = jnp.sum(x_tmp * x_tmp, axis=0)[None, :]
    return e_tmp, x_tmp, esum, esq, xsum, xsq


def _finalize_e(e_prev, e_tmp, esum, esq, bn_g2, bn_b2):
    mu = esum / float(B * N * N)
    var = esq / float(B * N * N) - mu * mu
    mu2 = _tile2(mu)
    inv2 = _tile2(jnp.sqrt(var + EPS))
    e_bn = bn_g2 * (e_tmp - mu2) / inv2 + bn_b2
    return e_prev + jnp.maximum(e_bn, 0.0)


def _finalize_x(x_prev, x_tmp, xsum, xsq, bn_g, bn_b):
    mu = xsum / float(B * N)
    var = xsq / float(B * N) - mu * mu
    x_bn = bn_g * (x_tmp - mu) / jnp.sqrt(var + EPS) + bn_b
    return x_prev + jnp.maximum(x_bn, 0.0)


def _finalize_x_pk(x_prev_pk, x_tmp_pk, xsum, xsq, bn_g2, bn_b2):
    mu = xsum / float(B * N)
    var = xsq / float(B * N) - mu * mu
    mu2 = _tile2(mu)
    inv2 = _tile2(jnp.sqrt(var + EPS))
    x_bn = bn_g2 * (x_tmp_pk - mu2) / inv2 + bn_b2
    return x_prev_pk + jnp.maximum(x_bn, 0.0)


def _acc(ref, val, first):
    @pl.when(first)
    def _():
        ref[...] = val

    @pl.when(jnp.logical_not(first))
    def _():
        ref[...] = ref[...] + val


def _k0_body(pk_ref, coord_ref, cpk_ref, wcoord_ref, wq4_ref, weval_ref,
             etag_ref,
             eU_w2, eU_b2, eV_wd, eV_w2, eV_b2, nU_w, nU_b, nV_w2, nV_b2,
             e0_ref, etmp_ref, x0_ref, xtmp_ref, esum_ref, esq_ref,
             xsum_ref, xsq_ref):
    first = pl.program_id(0) == 0
    coord = coord_ref[0]
    f32 = jnp.float32
    cq = coord.astype(jnp.bfloat16).astype(f32)
    wq = wcoord_ref[...].astype(jnp.bfloat16).astype(f32)
    x0 = cq[:, 0:1] * wq[0:1, :] + cq[:, 1:2] * wq[1:2, :]
    cpk = cpk_ref[0].astype(jnp.bfloat16).astype(f32)
    wq4 = wq4_ref[...]
    x0_pk = (cpk[:, 0:1] * wq4[0:1, :] + cpk[:, 1:2] * wq4[1:2, :]
             + cpk[:, 2:3] * wq4[2:3, :] + cpk[:, 3:4] * wq4[3:4, :])
    pk3 = jnp.repeat(pk_ref[0], H, axis=-1)
    tag = jnp.floor(pk3 * 0.25)
    evq3 = pk3 - 4.0 * tag
    wevq = weval_ref[...].astype(jnp.bfloat16).astype(f32)
    trow = jnp.where(tag == 0.0, etag_ref[0][None, None, :],
                     jnp.where(tag == 1.0, etag_ref[1][None, None, :],
                               etag_ref[2][None, None, :]))
    e0 = evq3 * wevq[0][None, None, :] + trow
    e0_ref[0] = e0
    x0_ref[0] = x0
    e_tmp, x_tmp, esum, esq, xsum, xsq = _layer_start(
        e0, x0, x0_pk, eU_w2[...], eU_b2[...], eV_wd[...], eV_w2[...],
        eV_b2[...], nU_w[...], nU_b[...], nV_w2[...], nV_b2[...])
    etmp_ref[0] = e_tmp
    xtmp_ref[0] = x_tmp
    _acc(esum_ref, esum, first)
    _acc(esq_ref, esq, first)
    _acc(xsum_ref, xsum, first)
    _acc(xsq_ref, xsq, first)


def _kmid_body(e_ref, etmp_prev_ref, x_ref, xpk_ref, xtmp_ref, xtmppk_ref,
               esum_ref, esq_ref, xsum_ref, xsq_ref,
               p_bn_e_g2, p_bn_e_b2, p_bn_n_g, p_bn_n_b, p_bn_n_g2,
               p_bn_n_b2,
               c_eU_w2, c_eU_b2, c_eV_wd, c_eV_w2, c_eV_b2, c_nU_w,
               c_nU_b, c_nV_w2, c_nV_b2,
               e_out_ref, etmp_out_ref, x_out_ref, xtmp_out_ref, esum_out,
               esq_out, xsum_out, xsq_out):
    first = pl.program_id(0) == 0
    e_prev = e_ref[0]
    x_prev = x_ref[0]
    x_new = _finalize_x(x_prev, xtmp_ref[0], xsum_ref[...], xsq_ref[...],
                        p_bn_n_g[...], p_bn_n_b[...])
    x_new_pk = _finalize_x_pk(xpk_ref[0], xtmppk_ref[0], xsum_ref[...],
                              xsq_ref[...], p_bn_n_g2[...], p_bn_n_b2[...])
    e_new = _finalize_e(e_prev, etmp_prev_ref[0], esum_ref[...],
                        esq_ref[...], p_bn_e_g2[...], p_bn_e_b2[...])
    e_out_ref[0] = e_new
    x_out_ref[0] = x_new
    e_tmp, x_tmp, esum, esq, xsum, xsq = _layer_start(
        e_new, x_new, x_new_pk, c_eU_w2[...], c_eU_b2[...], c_eV_wd[...],
        c_eV_w2[...], c_eV_b2[...], c_nU_w[...], c_nU_b[...],
        c_nV_w2[...], c_nV_b2[...])
    etmp_out_ref[0] = e_tmp
    xtmp_out_ref[0] = x_tmp
    _acc(esum_out, esum, first)
    _acc(esq_out, esq, first)
    _acc(xsum_out, xsum, first)
    _acc(xsq_out, xsq, first)


def _klast_body(e_ref, etmp_prev_ref, esum_ref, esq_ref,
                p_bn_e_g2, p_bn_e_b2,
                u_w2, u_b2, v_w4, v_b4,
                y_ref):
    e_new = _finalize_e(e_ref[0], etmp_prev_ref[0], esum_ref[...],
                        esq_ref[...], p_bn_e_g2[...], p_bn_e_b2[...])
    h = jnp.maximum(_mm(e_new, u_w2[...]) + u_b2[...][None], 0.0)
    y_ref[0] = _mm(h, v_w4[...]) + v_b4[...][None]


def _kloss_body(y_pk_ref, tgt_ref, acc_ref):
    first = pl.program_id(0) == 0
    y = y_pk_ref[0]
    tgt = tgt_ref[0]
    s0 = jnp.float32(0.0)
    s1 = jnp.float32(0.0)
    n1 = jnp.float32(0.0)
    for p in range(2):
        yp = y[:, :, 2 * p:2 * p + 2]
        m = jnp.max(yp, axis=-1, keepdims=True)
        lse = m + jnp.log(jnp.sum(jnp.exp(yp - m), axis=-1, keepdims=True))
        logp = yp - lse
        m1 = (tgt[:, :, p] == 1).astype(jnp.float32)
        s0 = s0 + jnp.sum(logp[:, :, 0] * (1.0 - m1))
        s1 = s1 + jnp.sum(logp[:, :, 1] * m1)
        n1 = n1 + jnp.sum(m1)
    lane = jax.lax.broadcasted_iota(jnp.int32, (1, 4), 1)
    vec = (jnp.where(lane == 0, s0, 0.0) + jnp.where(lane == 1, s1, 0.0)
           + jnp.where(lane == 2, n1, 0.0))
    _acc(acc_ref, vec, first)


def _full(x):
    nd = x.ndim
    return pl.BlockSpec(x.shape, lambda b, _n=nd: (0,) * _n)


def _bspec(shape):
    nd = len(shape)
    return pl.BlockSpec((1,) + shape[1:],
                        lambda b, _n=nd: (b,) + (0,) * (_n - 1))


@jax.jit
def _impl(edges, edges_values, nodes_coord, edges_target, params):
    f32 = jnp.float32
    wcoord = params['W_coord']
    weval_full = _tile2(jnp.concatenate(
        [params['W_eval'], jnp.zeros((1, H // 2), f32)], axis=1))
    etag_full = _tile2(jnp.concatenate(
        [jnp.zeros((3, H // 2), f32), params['E_tag']], axis=1))

    def bd(w):
        z = jnp.zeros_like(w)
        return jnp.concatenate(
            [jnp.concatenate([w, z], axis=1),
             jnp.concatenate([z, w], axis=1)], axis=0)

    r = lambda v: v.reshape(1, -1)

    def lay_start(l):
        p = params['layers'][l]
        return (bd(p['eU_w']), _tile2(r(p['eU_b'])),
                jnp.concatenate([p['eV_w'], p['eV_w']], axis=1),
                bd(p['eV_w']), _tile2(r(p['eV_b'])),
                p['nU_w'], r(p['nU_b']),
                bd(p['nV_w']), _tile2(r(p['nV_b'])))

    def lay_fin(l):
        p = params['layers'][l]
        return (_tile2(r(p['bn_e_g'])), _tile2(r(p['bn_e_b'])),
                r(p['bn_n_g']), r(p['bn_n_b']),
                _tile2(r(p['bn_n_g'])), _tile2(r(p['bn_n_b'])))

    pk = (edges_values.astype(jnp.bfloat16).astype(f32)
          + 4.0 * edges.astype(f32)).reshape(B, N, P2, 2)
    cpk = nodes_coord.reshape(B, P2, 4)
    wcq = wcoord.astype(jnp.bfloat16).astype(f32)
    zH = jnp.zeros((1, H), f32)
    wq4 = jnp.concatenate(
        [jnp.concatenate([wcq[0:1], zH], axis=1),
         jnp.concatenate([wcq[1:2], zH], axis=1),
         jnp.concatenate([zH, wcq[0:1]], axis=1),
         jnp.concatenate([zH, wcq[1:2]], axis=1)], axis=0)

    sH = jax.ShapeDtypeStruct((1, H), f32)
    eS = jax.ShapeDtypeStruct((B, N, P2, L), f32)
    xS = jax.ShapeDtypeStruct((B, N, H), f32)
    stat_spec = pl.BlockSpec((1, H), lambda b: (0, 0))

    L0 = lay_start(0)
    e0, etmp0, x0, xtmp0, esum0, esq0, xsum0, xsq0 = pl.pallas_call(
        _k0_body,
        grid=(B,),
        in_specs=[_bspec((B, N, P2, 2)), _bspec((B, N, 2)),
                  _bspec((B, P2, 4)), _full(wcoord), _full(wq4),
                  _full(weval_full), _full(etag_full)]
                 + [_full(w) for w in L0],
        out_specs=[_bspec((B, N, P2, L)), _bspec((B, N, P2, L)),
                   _bspec((B, N, H)), _bspec((B, N, H)), stat_spec,
                   stat_spec, stat_spec, stat_spec],
        out_shape=[eS, eS, xS, xS, sH, sH, sH, sH],
    )(pk, nodes_coord, cpk, wcoord, wq4, weval_full, etag_full, *L0)

    e, etmp, x, xtmp = e0, etmp0, x0, xtmp0
    esum, esq, xsum, xsq = esum0, esq0, xsum0, xsq0
    for l in range(1, NUM_LAYERS):
        prev_ops = lay_fin(l - 1)
        cur_ops = lay_start(l)
        x_pk = x.reshape(B, P2, L)
        xtmp_pk = xtmp.reshape(B, P2, L)
        e, etmp, x, xtmp, esum, esq, xsum, xsq = pl.pallas_call(
            _kmid_body,
            grid=(B,),
            in_specs=[_bspec((B, N, P2, L)), _bspec((B, N, P2, L)),
                      _bspec((B, N, H)), _bspec((B, P2, L)),
                      _bspec((B, N, H)), _bspec((B, P2, L)), stat_spec,
                      stat_spec, stat_spec, stat_spec]
                     + [_full(w) for w in prev_ops]
                     + [_full(w) for w in cur_ops],
            out_specs=[_bspec((B, N, P2, L)), _bspec((B, N, P2, L)),
                       _bspec((B, N, H)), _bspec((B, N, H)), stat_spec,
                       stat_spec, stat_spec, stat_spec],
            out_shape=[eS, eS, xS, xS, sH, sH, sH, sH],
        )(e, etmp, x, x_pk, xtmp, xtmp_pk, esum, esq, xsum, xsq,
          *prev_ops, *cur_ops)

    prev_ops = lay_fin(NUM_LAYERS - 1)[:2]
    u_w2 = bd(params['mlp_U'][0]['w'])
    u_b2 = _tile2(params['mlp_U'][0]['b'].reshape(1, H))
    vw = params['mlp_V_w']
    zv = jnp.zeros_like(vw)
    v_w4 = jnp.concatenate(
        [jnp.concatenate([vw, zv], axis=1),
         jnp.concatenate([zv, vw], axis=1)], axis=0)
    v_b4 = _tile2(params['mlp_V_b'].reshape(1, 2))
    tgt_pk = edges_target.reshape(B, N, P2, 2)
    y_pk = pl.pallas_call(
        _klast_body,
        grid=(B,),
        in_specs=[_bspec((B, N, P2, L)), _bspec((B, N, P2, L)),
                  stat_spec, stat_spec]
                 + [_full(w) for w in prev_ops]
                 + [_full(u_w2), _full(u_b2), _full(v_w4), _full(v_b4)],
        out_specs=_bspec((B, N, P2, 4)),
        out_shape=jax.ShapeDtypeStruct((B, N, P2, 4), f32),
    )(e, etmp, esum, esq, *prev_ops, u_w2, u_b2, v_w4, v_b4)

    acc = pl.pallas_call(
        _kloss_body,
        grid=(B,),
        in_specs=[_bspec((B, N, P2, 4)), _bspec((B, N, P2, 2))],
        out_specs=pl.BlockSpec((1, 4), lambda b: (0, 0)),
        out_shape=jax.ShapeDtypeStruct((1, 4), f32),
    )(y_pk, tgt_pk)

    y_pred = y_pk.reshape(B, N, N, 2)
    s0, s1, n1 = acc[0, 0], acc[0, 1], acc[0, 2]
    total = float(B * N * N)
    n0 = total - n1
    cw0 = total / (2.0 * n0)
    cw1 = total / (2.0 * n1)
    loss = -(cw0 * s0 + cw1 * s1) / (cw0 * n0 + cw1 * n1)
    return y_pred, loss


def kernel(edges, edges_values, nodes_coord, edges_target, params):
    return _impl(edges, edges_values, nodes_coord, edges_target, params)


# speedup vs baseline: 1.0899x; 1.0899x over previous
import functools

import jax
import jax.numpy as jnp
from jax.experimental import pallas as pl

B, N, H = 20, 100, 64
NUM_LAYERS = 3
EPS = 1e-5
def _mm(a3, w):
    r, c, h = a3.shape
    out = jax.lax.dot_general(a3.reshape(r * c, h), w,
                              (((1,), (0,)), ((), ())),
                              preferred_element_type=jnp.float32)
    return out.reshape(r, c, out.shape[-1])


def _mm2(a2, w):
    return jax.lax.dot_general(a2, w, (((1,), (0,)), ((), ())),
                               preferred_element_type=jnp.float32)


def _layer_start(e_cur, x_cur, eU_w, eU_b, eV_w, eV_b, nU_w, nU_b,
                 nV_w, nV_b):
    Vx = _mm2(x_cur, eV_w) + eV_b
    Ue = _mm(e_cur, eU_w) + eU_b[None]
    e_tmp = Ue + Vx[:, None, :] + Vx[None, :, :]
    gate = jax.nn.sigmoid(e_tmp)
    Vx2 = _mm2(x_cur, nV_w) + nV_b
    Ux = _mm2(x_cur, nU_w) + nU_b
    num = jnp.sum(gate * Vx2[None, :, :], axis=1)
    den = jnp.sum(gate, axis=1)
    x_tmp = Ux + num / (1e-20 + den)
    esum = jnp.sum(e_tmp, axis=(0, 1))[None, :]
    esq = jnp.sum(e_tmp * e_tmp, axis=(0, 1))[None, :]
    xsum = jnp.sum(x_tmp, axis=0)[None, :]
    xsq = jnp.sum(x_tmp * x_tmp, axis=0)[None, :]
    return e_tmp, x_tmp, esum, esq, xsum, xsq


def _finalize_e(e_prev, e_tmp, esum, esq, bn_g, bn_b):
    mu = esum / float(B * N * N)
    var = esq / float(B * N * N) - mu * mu
    e_bn = bn_g * (e_tmp - mu) / jnp.sqrt(var + EPS) + bn_b
    return e_prev + jnp.maximum(e_bn, 0.0)


def _finalize_x(x_prev, x_tmp, xsum, xsq, bn_g, bn_b):
    mu = xsum / float(B * N)
    var = xsq / float(B * N) - mu * mu
    x_bn = bn_g * (x_tmp - mu) / jnp.sqrt(var + EPS) + bn_b
    return x_prev + jnp.maximum(x_bn, 0.0)


def _acc(ref, val, first):
    @pl.when(first)
    def _():
        ref[...] = val

    @pl.when(jnp.logical_not(first))
    def _():
        ref[...] = ref[...] + val


def _k0_body(pk_ref, coord_ref, wcoord_ref, weval_ref, etag_ref,
             eU_w, eU_b, eV_w, eV_b, nU_w, nU_b, nV_w, nV_b,
             e0_ref, etmp_ref, x0_ref, xtmp_ref, esum_ref, esq_ref,
             xsum_ref, xsq_ref):
    first = pl.program_id(0) == 0
    coord = coord_ref[0]
    f32 = jnp.float32
    cq = coord.astype(jnp.bfloat16).astype(f32)
    wq = wcoord_ref[...].astype(jnp.bfloat16).astype(f32)
    x0 = cq[:, 0:1] * wq[0:1, :] + cq[:, 1:2] * wq[1:2, :]
    pk3 = pk_ref[0][:, :, None] * jnp.ones((1, 1, H), f32)
    tag = jnp.floor(pk3 * 0.25)
    evq3 = pk3 - 4.0 * tag
    wevq = weval_ref[0].astype(jnp.bfloat16).astype(f32)
    trow = jnp.where(tag == 0.0, etag_ref[0][None, None, :],
                     jnp.where(tag == 1.0, etag_ref[1][None, None, :],
                               etag_ref[2][None, None, :]))
    e0 = evq3 * wevq[None, None, :] + trow
    e0_ref[0] = e0
    x0_ref[0] = x0
    e_tmp, x_tmp, esum, esq, xsum, xsq = _layer_start(
        e0, x0, eU_w[...], eU_b[...], eV_w[...], eV_b[...],
        nU_w[...], nU_b[...], nV_w[...], nV_b[...])
    etmp_ref[0] = e_tmp
    xtmp_ref[0] = x_tmp
    _acc(esum_ref, esum, first)
    _acc(esq_ref, esq, first)
    _acc(xsum_ref, xsum, first)
    _acc(xsq_ref, xsq, first)


def _kmid_body(e_ref, etmp_prev_ref, x_ref, xtmp_ref, esum_ref, esq_ref,
               xsum_ref, xsq_ref,
               p_bn_e_g, p_bn_e_b, p_bn_n_g, p_bn_n_b,
               c_eU_w, c_eU_b, c_eV_w, c_eV_b, c_nU_w, c_nU_b, c_nV_w,
               c_nV_b,
               e_out_ref, etmp_out_ref, x_out_ref, xtmp_out_ref, esum_out,
               esq_out, xsum_out, xsq_out):
    first = pl.program_id(0) == 0
    e_prev = e_ref[0]
    x_prev = x_ref[0]
    x_new = _finalize_x(x_prev, xtmp_ref[0], xsum_ref[...], xsq_ref[...],
                        p_bn_n_g[...], p_bn_n_b[...])
    e_new = _finalize_e(e_prev, etmp_prev_ref[0], esum_ref[...],
                        esq_ref[...], p_bn_e_g[...], p_bn_e_b[...])
    e_out_ref[0] = e_new
    x_out_ref[0] = x_new
    e_tmp, x_tmp, esum, esq, xsum, xsq = _layer_start(
        e_new, x_new, c_eU_w[...], c_eU_b[...], c_eV_w[...], c_eV_b[...],
        c_nU_w[...], c_nU_b[...], c_nV_w[...], c_nV_b[...])
    etmp_out_ref[0] = e_tmp
    xtmp_out_ref[0] = x_tmp
    _acc(esum_out, esum, first)
    _acc(esq_out, esq, first)
    _acc(xsum_out, xsum, first)
    _acc(xsq_out, xsq, first)


def _klast_body(e_ref, etmp_prev_ref, esum_ref, esq_ref,
                p_bn_e_g, p_bn_e_b,
                u_w, u_b, v_w, v_b,
                y_ref):
    e_new = _finalize_e(e_ref[0], etmp_prev_ref[0], esum_ref[...],
                        esq_ref[...], p_bn_e_g[...], p_bn_e_b[...])
    h = jnp.maximum(_mm(e_new, u_w[...]) + u_b[...][None], 0.0)
    y_ref[0] = _mm(h, v_w[...]) + v_b[...][None]


def _kloss_body(y_ref, tgt_ref, acc_ref):
    first = pl.program_id(0) == 0
    y = y_ref[0]
    y0 = y[:, :, 0]
    y1 = y[:, :, 1]
    m = jnp.maximum(y0, y1)
    lse = m + jnp.log(jnp.exp(y0 - m) + jnp.exp(y1 - m))
    tgt = tgt_ref[0]
    mask1 = (tgt == 1).astype(jnp.float32)
    s0 = jnp.sum((y0 - lse) * (1.0 - mask1))
    s1 = jnp.sum((y1 - lse) * mask1)
    n1 = jnp.sum(mask1)
    lane = jax.lax.broadcasted_iota(jnp.int32, (1, 4), 1)
    vec = (jnp.where(lane == 0, s0, 0.0) + jnp.where(lane == 1, s1, 0.0)
           + jnp.where(lane == 2, n1, 0.0))
    _acc(acc_ref, vec, first)


def _full(x):
    nd = x.ndim
    return pl.BlockSpec(x.shape, lambda b, _n=nd: (0,) * _n)


def _bspec(shape):
    nd = len(shape)
    return pl.BlockSpec((1,) + shape[1:],
                        lambda b, _n=nd: (b,) + (0,) * (_n - 1))


@jax.jit
def _impl(edges, edges_values, nodes_coord, edges_target, params):
    f32 = jnp.float32
    wcoord = params['W_coord']
    weval_full = jnp.concatenate(
        [params['W_eval'], jnp.zeros((1, H // 2), f32)], axis=1)
    etag_full = jnp.concatenate(
        [jnp.zeros((3, H // 2), f32), params['E_tag']], axis=1)

    def lay(l):
        p = params['layers'][l]
        r = lambda v: v.reshape(1, -1)
        return (p['eU_w'], r(p['eU_b']), p['eV_w'], r(p['eV_b']),
                p['nU_w'], r(p['nU_b']), p['nV_w'], r(p['nV_b']),
                r(p['bn_e_g']), r(p['bn_e_b']), r(p['bn_n_g']),
                r(p['bn_n_b']))

    sH = jax.ShapeDtypeStruct((1, H), f32)
    eS = jax.ShapeDtypeStruct((B, N, N, H), f32)
    xS = jax.ShapeDtypeStruct((B, N, H), f32)
    stat_spec = pl.BlockSpec((1, H), lambda b: (0, 0))

    pk = (edges_values.astype(jnp.bfloat16).astype(f32)
          + 4.0 * edges.astype(f32))

    L0 = lay(0)
    e0, etmp0, x0, xtmp0, esum0, esq0, xsum0, xsq0 = pl.pallas_call(
        _k0_body,
        grid=(B,),
        in_specs=[_bspec((B, N, N)), _bspec((B, N, 2)),
                  _full(wcoord), _full(weval_full), _full(etag_full)]
                 + [_full(w) for w in L0[:8]],
        out_specs=[_bspec((B, N, N, H)), _bspec((B, N, N, H)),
                   _bspec((B, N, H)), _bspec((B, N, H)), stat_spec,
                   stat_spec, stat_spec, stat_spec],
        out_shape=[eS, eS, xS, xS, sH, sH, sH, sH],
    )(pk, nodes_coord, wcoord, weval_full, etag_full, *L0[:8])

    e, etmp, x, xtmp = e0, etmp0, x0, xtmp0
    esum, esq, xsum, xsq = esum0, esq0, xsum0, xsq0
    for l in range(1, NUM_LAYERS):
        P, C = lay(l - 1), lay(l)
        prev_ops = (P[8], P[9], P[10], P[11])
        cur_ops = C[:8]
        e, etmp, x, xtmp, esum, esq, xsum, xsq = pl.pallas_call(
            _kmid_body,
            grid=(B,),
            in_specs=[_bspec((B, N, N, H)), _bspec((B, N, N, H)),
                      _bspec((B, N, H)), _bspec((B, N, H)), stat_spec,
                      stat_spec, stat_spec, stat_spec]
                     + [_full(w) for w in prev_ops]
                     + [_full(w) for w in cur_ops],
            out_specs=[_bspec((B, N, N, H)), _bspec((B, N, N, H)),
                       _bspec((B, N, H)), _bspec((B, N, H)), stat_spec,
                       stat_spec, stat_spec, stat_spec],
            out_shape=[eS, eS, xS, xS, sH, sH, sH, sH],
        )(e, etmp, x, xtmp, esum, esq, xsum, xsq, *prev_ops, *cur_ops)

    P = lay(NUM_LAYERS - 1)
    prev_ops = (P[8], P[9])
    u_w = params['mlp_U'][0]['w']
    u_b = params['mlp_U'][0]['b'].reshape(1, H)
    v_w = params['mlp_V_w']
    v_b = params['mlp_V_b'].reshape(1, 2)
    y_pred = pl.pallas_call(
        _klast_body,
        grid=(B,),
        in_specs=[_bspec((B, N, N, H)), _bspec((B, N, N, H)),
                  stat_spec, stat_spec]
                 + [_full(w) for w in prev_ops]
                 + [_full(u_w), _full(u_b), _full(v_w), _full(v_b)],
        out_specs=_bspec((B, N, N, 2)),
        out_shape=jax.ShapeDtypeStruct((B, N, N, 2), f32),
    )(e, etmp, esum, esq, *prev_ops, u_w, u_b, v_w, v_b)

    acc = pl.pallas_call(
        _kloss_body,
        grid=(B,),
        in_specs=[_bspec((B, N, N, 2)), _bspec((B, N, N))],
        out_specs=pl.BlockSpec((1, 4), lambda b: (0, 0)),
        out_shape=jax.ShapeDtypeStruct((1, 4), f32),
    )(y_pred, edges_target)

    s0, s1, n1 = acc[0, 0], acc[0, 1], acc[0, 2]
    total = float(B * N * N)
    n0 = total - n1
    cw0 = total / (2.0 * n0)
    cw1 = total / (2.0 * n1)
    loss = -(cw0 * s0 + cw1 * s1) / (cw0 * n0 + cw1 * n1)
    return y_pred, loss


def kernel(edges, edges_values, nodes_coord, edges_target, params):
    return _impl(edges, edges_values, nodes_coord, edges_target, params)


# speedup vs baseline: 1.2288x; 1.1274x over previous
import functools

import jax
import jax.numpy as jnp
from jax.experimental import pallas as pl

B, N, H = 20, 100, 64
NUM_LAYERS = 3
EPS = 1e-5
def _mm(a3, w):
    r, c, h = a3.shape
    out = jax.lax.dot_general(a3.reshape(r * c, h), w,
                              (((1,), (0,)), ((), ())),
                              preferred_element_type=jnp.float32)
    return out.reshape(r, c, out.shape[-1])


def _mm2(a2, w):
    return jax.lax.dot_general(a2, w, (((1,), (0,)), ((), ())),
                               preferred_element_type=jnp.float32)


def _layer_start(e_cur, x_cur, eU_w, eU_b, eV_w, eV_b, nU_w, nU_b,
                 nV_w, nV_b):
    Vx = _mm2(x_cur, eV_w) + eV_b
    Ue = _mm(e_cur, eU_w) + eU_b[None]
    e_tmp = Ue + Vx[:, None, :] + Vx[None, :, :]
    gate = jax.nn.sigmoid(e_tmp)
    Vx2 = _mm2(x_cur, nV_w) + nV_b
    Ux = _mm2(x_cur, nU_w) + nU_b
    num = jnp.sum(gate * Vx2[None, :, :], axis=1)
    den = jnp.sum(gate, axis=1)
    x_tmp = Ux + num / (1e-20 + den)
    esum = jnp.sum(e_tmp, axis=(0, 1))[None, :]
    esq = jnp.sum(e_tmp * e_tmp, axis=(0, 1))[None, :]
    xsum = jnp.sum(x_tmp, axis=0)[None, :]
    xsq = jnp.sum(x_tmp * x_tmp, axis=0)[None, :]
    return e_tmp, x_tmp, esum, esq, xsum, xsq


def _finalize_e(e_prev, e_tmp, esum, esq, bn_g, bn_b):
    mu = esum / float(B * N * N)
    var = esq / float(B * N * N) - mu * mu
    e_bn = bn_g * (e_tmp - mu) / jnp.sqrt(var + EPS) + bn_b
    return e_prev + jnp.maximum(e_bn, 0.0)


def _finalize_x(x_prev, x_tmp, xsum, xsq, bn_g, bn_b):
    mu = xsum / float(B * N)
    var = xsq / float(B * N) - mu * mu
    x_bn = bn_g * (x_tmp - mu) / jnp.sqrt(var + EPS) + bn_b
    return x_prev + jnp.maximum(x_bn, 0.0)


def _acc(ref, val, first):
    @pl.when(first)
    def _():
        ref[...] = val

    @pl.when(jnp.logical_not(first))
    def _():
        ref[...] = ref[...] + val


def _k0_body(pk_ref, coord_ref, wcoord_ref, weval_ref, etag_ref,
             eU_w, eU_b, eV_w, eV_b, nU_w, nU_b, nV_w, nV_b,
             e0_ref, etmp_ref, x0_ref, xtmp_ref, esum_ref, esq_ref,
             xsum_ref, xsq_ref):
    first = pl.program_id(0) == 0
    coord = coord_ref[0]
    f32 = jnp.float32
    cq = coord.astype(jnp.bfloat16).astype(f32)
    wq = wcoord_ref[...].astype(jnp.bfloat16).astype(f32)
    x0 = cq[:, 0:1] * wq[0:1, :] + cq[:, 1:2] * wq[1:2, :]
    pk3 = pk_ref[0][:, :, None] * jnp.ones((1, 1, H), f32)
    tag = jnp.floor(pk3 * 0.25)
    evq3 = pk3 - 4.0 * tag
    wevq = weval_ref[0].astype(jnp.bfloat16).astype(f32)
    trow = jnp.where(tag == 0.0, etag_ref[0][None, None, :],
                     jnp.where(tag == 1.0, etag_ref[1][None, None, :],
                               etag_ref[2][None, None, :]))
    e0 = evq3 * wevq[None, None, :] + trow
    e0_ref[0] = e0
    x0_ref[0] = x0
    e_tmp, x_tmp, esum, esq, xsum, xsq = _layer_start(
        e0, x0, eU_w[...], eU_b[...], eV_w[...], eV_b[...],
        nU_w[...], nU_b[...], nV_w[...], nV_b[...])
    etmp_ref[0] = e_tmp
    xtmp_ref[0] = x_tmp
    _acc(esum_ref, esum, first)
    _acc(esq_ref, esq, first)
    _acc(xsum_ref, xsum, first)
    _acc(xsq_ref, xsq, first)


def _kmid_body(e_ref, etmp_prev_ref, x_ref, xtmp_ref, esum_ref, esq_ref,
               xsum_ref, xsq_ref,
               p_bn_e_g, p_bn_e_b, p_bn_n_g, p_bn_n_b,
               c_eU_w, c_eU_b, c_eV_w, c_eV_b, c_nU_w, c_nU_b, c_nV_w,
               c_nV_b,
               e_out_ref, etmp_out_ref, x_out_ref, xtmp_out_ref, esum_out,
               esq_out, xsum_out, xsq_out):
    first = pl.program_id(0) == 0
    e_prev = e_ref[0]
    x_prev = x_ref[0]
    x_new = _finalize_x(x_prev, xtmp_ref[0], xsum_ref[...], xsq_ref[...],
                        p_bn_n_g[...], p_bn_n_b[...])
    e_new = _finalize_e(e_prev, etmp_prev_ref[0], esum_ref[...],
                        esq_ref[...], p_bn_e_g[...], p_bn_e_b[...])
    e_out_ref[0] = e_new
    x_out_ref[0] = x_new
    e_tmp, x_tmp, esum, esq, xsum, xsq = _layer_start(
        e_new, x_new, c_eU_w[...], c_eU_b[...], c_eV_w[...], c_eV_b[...],
        c_nU_w[...], c_nU_b[...], c_nV_w[...], c_nV_b[...])
    etmp_out_ref[0] = e_tmp
    xtmp_out_ref[0] = x_tmp
    _acc(esum_out, esum, first)
    _acc(esq_out, esq, first)
    _acc(xsum_out, xsum, first)
    _acc(xsq_out, xsq, first)


def _klast_body(e_ref, etmp_prev_ref, esum_ref, esq_ref, tgt_ref,
                p_bn_e_g, p_bn_e_b,
                u_w, u_b, v_w, v_b,
                y_ref, acc_ref):
    first = pl.program_id(0) == 0
    e_prev = e_ref[0]
    e_new = _finalize_e(e_prev, etmp_prev_ref[0], esum_ref[...],
                        esq_ref[...], p_bn_e_g[...], p_bn_e_b[...])
    h = jnp.maximum(_mm(e_new, u_w[...]) + u_b[...][None], 0.0)
    y = _mm(h, v_w[...]) + v_b[...][None]
    y_ref[0] = y
    m = jnp.max(y, axis=-1, keepdims=True)
    lse = m + jnp.log(jnp.sum(jnp.exp(y - m), axis=-1, keepdims=True))
    logp = y - lse
    tgt = tgt_ref[0]
    mask1 = (tgt == 1).astype(jnp.float32)
    mask0 = 1.0 - mask1
    s0 = jnp.sum(logp[:, :, 0] * mask0)
    s1 = jnp.sum(logp[:, :, 1] * mask1)
    n1 = jnp.sum(mask1)
    lane = jax.lax.broadcasted_iota(jnp.int32, (1, 4), 1)
    vec = (jnp.where(lane == 0, s0, 0.0) + jnp.where(lane == 1, s1, 0.0)
           + jnp.where(lane == 2, n1, 0.0))
    _acc(acc_ref, vec, first)


def _full(x):
    nd = x.ndim
    return pl.BlockSpec(x.shape, lambda b, _n=nd: (0,) * _n)


def _bspec(shape):
    nd = len(shape)
    return pl.BlockSpec((1,) + shape[1:],
                        lambda b, _n=nd: (b,) + (0,) * (_n - 1))


@jax.jit
def _impl(edges, edges_values, nodes_coord, edges_target, params):
    f32 = jnp.float32
    wcoord = params['W_coord']
    weval_full = jnp.concatenate(
        [params['W_eval'], jnp.zeros((1, H // 2), f32)], axis=1)
    etag_full = jnp.concatenate(
        [jnp.zeros((3, H // 2), f32), params['E_tag']], axis=1)

    def lay(l):
        p = params['layers'][l]
        r = lambda v: v.reshape(1, -1)
        return (p['eU_w'], r(p['eU_b']), p['eV_w'], r(p['eV_b']),
                p['nU_w'], r(p['nU_b']), p['nV_w'], r(p['nV_b']),
                r(p['bn_e_g']), r(p['bn_e_b']), r(p['bn_n_g']),
                r(p['bn_n_b']))

    sH = jax.ShapeDtypeStruct((1, H), f32)
    eS = jax.ShapeDtypeStruct((B, N, N, H), f32)
    xS = jax.ShapeDtypeStruct((B, N, H), f32)
    stat_spec = pl.BlockSpec((1, H), lambda b: (0, 0))

    pk = (edges_values.astype(jnp.bfloat16).astype(f32)
          + 4.0 * edges.astype(f32))

    L0 = lay(0)
    e0, etmp0, x0, xtmp0, esum0, esq0, xsum0, xsq0 = pl.pallas_call(
        _k0_body,
        grid=(B,),
        in_specs=[_bspec((B, N, N)), _bspec((B, N, 2)),
                  _full(wcoord), _full(weval_full), _full(etag_full)]
                 + [_full(w) for w in L0[:8]],
        out_specs=[_bspec((B, N, N, H)), _bspec((B, N, N, H)),
                   _bspec((B, N, H)), _bspec((B, N, H)), stat_spec,
                   stat_spec, stat_spec, stat_spec],
        out_shape=[eS, eS, xS, xS, sH, sH, sH, sH],
    )(pk, nodes_coord, wcoord, weval_full, etag_full, *L0[:8])

    e, etmp, x, xtmp = e0, etmp0, x0, xtmp0
    esum, esq, xsum, xsq = esum0, esq0, xsum0, xsq0
    for l in range(1, NUM_LAYERS):
        P, C = lay(l - 1), lay(l)
        prev_ops = (P[8], P[9], P[10], P[11])
        cur_ops = C[:8]
        e, etmp, x, xtmp, esum, esq, xsum, xsq = pl.pallas_call(
            _kmid_body,
            grid=(B,),
            in_specs=[_bspec((B, N, N, H)), _bspec((B, N, N, H)),
                      _bspec((B, N, H)), _bspec((B, N, H)), stat_spec,
                      stat_spec, stat_spec, stat_spec]
                     + [_full(w) for w in prev_ops]
                     + [_full(w) for w in cur_ops],
            out_specs=[_bspec((B, N, N, H)), _bspec((B, N, N, H)),
                       _bspec((B, N, H)), _bspec((B, N, H)), stat_spec,
                       stat_spec, stat_spec, stat_spec],
            out_shape=[eS, eS, xS, xS, sH, sH, sH, sH],
        )(e, etmp, x, xtmp, esum, esq, xsum, xsq, *prev_ops, *cur_ops)

    P = lay(NUM_LAYERS - 1)
    prev_ops = (P[8], P[9])
    u_w = params['mlp_U'][0]['w']
    u_b = params['mlp_U'][0]['b'].reshape(1, H)
    v_w = params['mlp_V_w']
    v_b = params['mlp_V_b'].reshape(1, 2)
    y_pred, acc = pl.pallas_call(
        _klast_body,
        grid=(B,),
        in_specs=[_bspec((B, N, N, H)), _bspec((B, N, N, H)),
                  stat_spec, stat_spec, _bspec((B, N, N))]
                 + [_full(w) for w in prev_ops]
                 + [_full(u_w), _full(u_b), _full(v_w), _full(v_b)],
        out_specs=[_bspec((B, N, N, 2)),
                   pl.BlockSpec((1, 4), lambda b: (0, 0))],
        out_shape=[jax.ShapeDtypeStruct((B, N, N, 2), f32),
                   jax.ShapeDtypeStruct((1, 4), f32)],
    )(e, etmp, esum, esq, edges_target, *prev_ops, u_w, u_b, v_w, v_b)

    s0, s1, n1 = acc[0, 0], acc[0, 1], acc[0, 2]
    total = float(B * N * N)
    n0 = total - n1
    cw0 = total / (2.0 * n0)
    cw1 = total / (2.0 * n1)
    loss = -(cw0 * s0 + cw1 * s1) / (cw0 * n0 + cw1 * n1)
    return y_pred, loss


def kernel(edges, edges_values, nodes_coord, edges_target, params):
    return _impl(edges, edges_values, nodes_coord, edges_target, params)
